# Initial kernel scaffold; baseline (speedup 1.0000x reference)
import jax
import jax.numpy as jnp
from jax.experimental import pallas as pl


def kernel(node_ids, neighbor_ids, path_ids, edge_ids, ent_table, rel_table):
    raise NotImplementedError("write your pallas kernel here")


# speedup vs baseline: 1.6944x; 1.6944x over previous
import functools
import math

import jax
import jax.numpy as jnp
from jax import lax
from jax.experimental import pallas as pl
from jax.experimental.pallas import tpu as pltpu
from jax.experimental.pallas import tpu_sc as plsc

DIM = 64
LN = 32
LP = 32
LE = 16
NC = 2
NS = 16
NW = NC * NS
QUAD = 4

_LN2 = math.log(2.0)


_GATHER_DNUMS = lax.GatherDimensionNumbers(
    offset_dims=(), collapsed_slice_dims=(0,), start_index_map=(0,))


def _shuffle(v, idx):
    return lax.gather(v, idx[:, None], _GATHER_DNUMS, slice_sizes=(1,),
                      mode=lax.GatherScatterMode.PROMISE_IN_BOUNDS)


def _lane_sum(v):
    for sh in (8, 4, 2, 1):
        idx = lax.iota(jnp.int32, 16) ^ sh
        v = v + _shuffle(v, idx)
    return v


def _rsqrt(x):
    i = plsc.bitcast(x, jnp.int32)
    y = plsc.bitcast(jnp.int32(0x5F3759DF) - lax.shift_right_arithmetic(i, 1),
                     jnp.float32)
    for _ in range(3):
        y = y * (jnp.float32(1.5) - jnp.float32(0.5) * x * y * y)
    return y


def _log(x):
    i = plsc.bitcast(x, jnp.int32)
    f = i.astype(jnp.float32)
    y = f * jnp.float32(_LN2 / (1 << 23)) - jnp.float32(126.94269504 * _LN2)
    for _ in range(3):
        y = y + x * jnp.exp(-y) - jnp.float32(1.0)
    return y


def _ctx(ref, rb, L, si):
    s0, s1, s2, s3 = si
    q0 = s0 * s0 + s1 * s1 + s2 * s2 + s3 * s3

    def p1(i, carry):
        a0, a1, a2, a3, qa = carry
        r = rb + i
        r0 = ref[r, pl.ds(0, 16)]
        r1 = ref[r, pl.ds(16, 16)]
        r2 = ref[r, pl.ds(32, 16)]
        r3 = ref[r, pl.ds(48, 16)]
        return (a0 + r0, a1 + r1, a2 + r2, a3 + r3,
                qa + r0 * r0 + r1 * r1 + r2 * r2 + r3 * r3)

    a0, a1, a2, a3, qa = lax.fori_loop(0, L, p1, (s0, s1, s2, s3, q0),
                                       unroll=4)
    q = _lane_sum(qa)
    rinv = _rsqrt(q)
    g0, g1, g2, g3 = a0 * rinv, a1 * rinv, a2 * rinv, a3 * rinv
    sc0 = _lane_sum(s0 * g0 + s1 * g1 + s2 * g2 + s3 * g3)
    sum_sc = _lane_sum(a0 * g0 + a1 * g1 + a2 * g2 + a3 * g3)

    def p2(i, se):
        r = rb + i
        r0 = ref[r, pl.ds(0, 16)]
        r1 = ref[r, pl.ds(16, 16)]
        r2 = ref[r, pl.ds(32, 16)]
        r3 = ref[r, pl.ds(48, 16)]
        sc = _lane_sum(r0 * g0 + r1 * g1 + r2 * g2 + r3 * g3)
        return se + jnp.exp(sc)

    se = lax.fori_loop(0, L, p2, jnp.exp(sc0), unroll=4)
    lse = _log(se)
    return sum_sc - jnp.float32(L + 1) * lse, sc0 - lse


@functools.lru_cache(maxsize=None)
def _build(B):
    assert B % NW == 0
    SPW = B // NW
    NQ = SPW // QUAD

    mesh = plsc.VectorSubcoreMesh(core_axis_name="c", subcore_axis_name="s",
                                  num_cores=NC, num_subcores=NS)

    @functools.partial(
        pl.kernel,
        out_type=jax.ShapeDtypeStruct((B,), jnp.float32),
        mesh=mesh,
        compiler_params=pltpu.CompilerParams(needs_layout_passes=False,
                                             use_tc_tiling_on_sc=False),
        scratch_types=[
            pltpu.VMEM((SPW,), jnp.int32),
            pltpu.VMEM((SPW * LN,), jnp.int32),
            pltpu.VMEM((SPW * LP,), jnp.int32),
            pltpu.VMEM((SPW * LE,), jnp.int32),
            pltpu.VMEM((SPW, DIM), jnp.float32),
            pltpu.VMEM((QUAD * LN, DIM), jnp.float32),
            pltpu.VMEM((QUAD * LP, DIM), jnp.float32),
            pltpu.VMEM((QUAD * LE, DIM), jnp.float32),
            pltpu.VMEM((SPW,), jnp.float32),
            pltpu.SemaphoreType.DMA,
        ],
    )
    def sc_kernel(nid_h, nbr_h, pth_h, edg_h, ent_h, rel_h, out_h,
                  nid_v, nbr_v, pth_v, edg_v, si_all, n_r, p_r, e_r,
                  out_v, sem):
        c = lax.axis_index("c")
        s = lax.axis_index("s")
        wid = s * NC + c
        base = wid * SPW

        pltpu.sync_copy(nid_h.at[pl.ds(base, SPW)], nid_v)
        pltpu.sync_copy(nbr_h.at[pl.ds(base * LN, SPW * LN)], nbr_v)
        pltpu.sync_copy(pth_h.at[pl.ds(base * LP, SPW * LP)], pth_v)
        pltpu.sync_copy(edg_h.at[pl.ds(base * LE, SPW * LE)], edg_v)

        pltpu.async_copy(ent_h.at[nid_v], si_all, sem).wait()

        lane0 = lax.iota(jnp.int32, 16) == 0

        def quad(q, _):
            d1 = pltpu.async_copy(
                ent_h.at[nbr_v.at[pl.ds(q * (QUAD * LN), QUAD * LN)]],
                n_r, sem)
            d2 = pltpu.async_copy(
                ent_h.at[pth_v.at[pl.ds(q * (QUAD * LP), QUAD * LP)]],
                p_r, sem)
            d3 = pltpu.async_copy(
                rel_h.at[edg_v.at[pl.ds(q * (QUAD * LE), QUAD * LE)]],
                e_r, sem)
            d1.wait()
            d2.wait()
            d3.wait()
            for j in range(QUAD):
                t = q * QUAD + j
                si = tuple(si_all[t, pl.ds(16 * k, 16)] for k in range(4))
                n_acc, _ = _ctx(n_r, j * LN, LN, si)
                p_acc, _ = _ctx(p_r, j * LP, LP, si)
                _, e_first = _ctx(e_r, j * LE, LE, si)
                loss = -(n_acc + jnp.float32(0.1) * p_acc
                         + jnp.float32(0.1) * e_first)
                idx = lax.broadcast(t, (16,)).astype(jnp.int32)
                plsc.store_scatter(out_v, [idx], loss, mask=lane0)
            return 0

        lax.fori_loop(0, NQ, quad, 0)
        pltpu.sync_copy(out_v, out_h.at[pl.ds(base, SPW)])

    return sc_kernel


def kernel(node_ids, neighbor_ids, path_ids, edge_ids, ent_table, rel_table):
    B = node_ids.shape[0]
    f = _build(B)
    return f(node_ids.astype(jnp.int32),
             neighbor_ids.astype(jnp.int32).reshape(-1),
             path_ids.astype(jnp.int32).reshape(-1),
             edge_ids.astype(jnp.int32).reshape(-1),
             ent_table, rel_table)


# speedup vs baseline: 1.7189x; 1.0145x over previous
import functools
import math

import jax
import jax.numpy as jnp
from jax import lax
from jax.experimental import pallas as pl
from jax.experimental.pallas import tpu as pltpu
from jax.experimental.pallas import tpu_sc as plsc

DIM = 64
LN = 32
LP = 32
LE = 16
NC = 2
NS = 16
NW = NC * NS
QUAD = 4
NBUF = 2

_LN2 = math.log(2.0)

_GATHER_DNUMS = lax.GatherDimensionNumbers(
    offset_dims=(), collapsed_slice_dims=(0,), start_index_map=(0,))


def _shuffle(v, idx):
    return lax.gather(v, idx[:, None], _GATHER_DNUMS, slice_sizes=(1,),
                      mode=lax.GatherScatterMode.PROMISE_IN_BOUNDS)


def _lane_sum(v):
    for sh in (8, 4, 2, 1):
        idx = lax.iota(jnp.int32, 16) ^ sh
        v = v + _shuffle(v, idx)
    return v


def _rsqrt(x):
    i = plsc.bitcast(x, jnp.int32)
    y = plsc.bitcast(jnp.int32(0x5F3759DF) - lax.shift_right_arithmetic(i, 1),
                     jnp.float32)
    for _ in range(3):
        y = y * (jnp.float32(1.5) - jnp.float32(0.5) * x * y * y)
    return y


def _log(x):
    i = plsc.bitcast(x, jnp.int32)
    f = i.astype(jnp.float32)
    y = f * jnp.float32(_LN2 / (1 << 23)) - jnp.float32(126.94269504 * _LN2)
    for _ in range(3):
        y = y + x * jnp.exp(-y) - jnp.float32(1.0)
    return y


def _ctx(row_load, L, si, unroll):
    s0, s1, s2, s3 = si
    q0 = s0 * s0 + s1 * s1 + s2 * s2 + s3 * s3

    def p1(i, carry):
        a0, a1, a2, a3, qa = carry
        r0, r1, r2, r3 = row_load(i)
        return (a0 + r0, a1 + r1, a2 + r2, a3 + r3,
                qa + r0 * r0 + r1 * r1 + r2 * r2 + r3 * r3)

    a0, a1, a2, a3, qa = lax.fori_loop(0, L, p1, (s0, s1, s2, s3, q0),
                                       unroll=unroll)
    q = _lane_sum(qa)
    rinv = _rsqrt(q)
    g0, g1, g2, g3 = a0 * rinv, a1 * rinv, a2 * rinv, a3 * rinv
    sc0 = _lane_sum(s0 * g0 + s1 * g1 + s2 * g2 + s3 * g3)
    sum_sc = _lane_sum(a0 * g0 + a1 * g1 + a2 * g2 + a3 * g3)

    def p2(i, se):
        r0, r1, r2, r3 = row_load(i)
        sc = _lane_sum(r0 * g0 + r1 * g1 + r2 * g2 + r3 * g3)
        return se + jnp.exp(sc)

    se = lax.fori_loop(0, L, p2, jnp.exp(sc0), unroll=unroll)
    lse = _log(se)
    return sum_sc - jnp.float32(L + 1) * lse, sc0 - lse


@functools.lru_cache(maxsize=None)
def _build(B):
    assert B % NW == 0
    SPW = B // NW
    NQ = SPW // QUAD

    mesh = plsc.VectorSubcoreMesh(core_axis_name="c", subcore_axis_name="s",
                                  num_cores=NC, num_subcores=NS)

    @functools.partial(
        pl.kernel,
        out_type=jax.ShapeDtypeStruct((B,), jnp.float32),
        mesh=mesh,
        compiler_params=pltpu.CompilerParams(needs_layout_passes=False,
                                             use_tc_tiling_on_sc=False),
        scratch_types=[
            pltpu.VMEM((SPW,), jnp.int32),
            pltpu.VMEM((SPW * LN,), jnp.int32),
            pltpu.VMEM((SPW * LP,), jnp.int32),
            pltpu.VMEM((SPW * LE,), jnp.int32),
            pltpu.VMEM((SPW, DIM), jnp.float32),
            pltpu.VMEM((NBUF * QUAD * LN, DIM), jnp.float32),
            pltpu.VMEM((NBUF * QUAD * LP, DIM), jnp.float32),
            pltpu.VMEM((NBUF * QUAD * LE, DIM), jnp.float32),
            pltpu.VMEM((SPW,), jnp.float32),
            pltpu.SemaphoreType.DMA,
            pltpu.SemaphoreType.DMA,
        ],
    )
    def sc_kernel(nid_h, nbr_h, pth_h, edg_h, ent_h, rel_h, out_h,
                  nid_v, nbr_v, pth_v, edg_v, si_all, n_r, p_r, e_r,
                  out_v, sem0, sem1):
        c = lax.axis_index("c")
        s = lax.axis_index("s")
        wid = s * NC + c
        base = wid * SPW

        pltpu.sync_copy(nid_h.at[pl.ds(base, SPW)], nid_v)
        pltpu.sync_copy(nbr_h.at[pl.ds(base * LN, SPW * LN)], nbr_v)
        pltpu.sync_copy(pth_h.at[pl.ds(base * LP, SPW * LP)], pth_v)
        pltpu.sync_copy(edg_h.at[pl.ds(base * LE, SPW * LE)], edg_v)

        pltpu.async_copy(ent_h.at[nid_v], si_all, sem0).wait()

        def dmas(q, slot, sem):
            return (
                pltpu.make_async_copy(
                    ent_h.at[nbr_v.at[pl.ds(q * (QUAD * LN), QUAD * LN)]],
                    n_r.at[pl.ds(slot * (QUAD * LN), QUAD * LN)], sem),
                pltpu.make_async_copy(
                    ent_h.at[pth_v.at[pl.ds(q * (QUAD * LP), QUAD * LP)]],
                    p_r.at[pl.ds(slot * (QUAD * LP), QUAD * LP)], sem),
                pltpu.make_async_copy(
                    rel_h.at[edg_v.at[pl.ds(q * (QUAD * LE), QUAD * LE)]],
                    e_r.at[pl.ds(slot * (QUAD * LE), QUAD * LE)], sem),
            )

        def issue(q, slot, sem):
            for d in dmas(q, slot, sem):
                d.start()

        def drain(q, slot, sem):
            for d in dmas(q, slot, sem):
                d.wait()

        issue(0, 0, sem0)

        lane = lax.iota(jnp.int32, 16)
        lane0 = lane == 0

        def quad(q, _):
            par = jnp.bitwise_and(q, 1)
            cur = par
            nxt = 1 - par

            @pl.when(jnp.logical_and(q + 1 < NQ, par == 0))
            def _():
                issue(q + 1, 1, sem1)

            @pl.when(jnp.logical_and(q + 1 < NQ, par == 1))
            def _():
                issue(q + 1, 0, sem0)

            @pl.when(par == 0)
            def _():
                drain(q, 0, sem0)

            @pl.when(par == 1)
            def _():
                drain(q, 1, sem1)

            slot0 = cur * QUAD
            for j in range(QUAD):
                t = q * QUAD + j
                nb = (slot0 + j) * LN
                pb = (slot0 + j) * LP
                eb = (slot0 + j) * LE
                si = tuple(si_all[t, pl.ds(16 * k, 16)] for k in range(4))
                n_acc, _ = _ctx(
                    lambda i: tuple(n_r[nb + i, pl.ds(16 * k, 16)]
                                    for k in range(4)), LN, si, 8)
                p_acc, _ = _ctx(
                    lambda i: tuple(p_r[pb + i, pl.ds(16 * k, 16)]
                                    for k in range(4)), LP, si, 8)
                _, e_first = _ctx(
                    lambda i: tuple(e_r[eb + i, pl.ds(16 * k, 16)]
                                    for k in range(4)), LE, si, 8)
                loss = -(n_acc + jnp.float32(0.1) * p_acc
                         + jnp.float32(0.1) * e_first)
                idx = lax.broadcast(t, (16,)).astype(jnp.int32)
                plsc.store_scatter(out_v, [idx], loss, mask=lane0)
            return 0

        lax.fori_loop(0, NQ, quad, 0)
        pltpu.sync_copy(out_v, out_h.at[pl.ds(base, SPW)])

    return sc_kernel


def kernel(node_ids, neighbor_ids, path_ids, edge_ids, ent_table, rel_table):
    B = node_ids.shape[0]
    f = _build(B)
    return f(node_ids.astype(jnp.int32),
             neighbor_ids.astype(jnp.int32).reshape(-1),
             path_ids.astype(jnp.int32).reshape(-1),
             edge_ids.astype(jnp.int32).reshape(-1),
             ent_table, rel_table)


# speedup vs baseline: 2.0006x; 1.1639x over previous
import functools
import math

import jax
import jax.numpy as jnp
from jax import lax
from jax.experimental import pallas as pl
from jax.experimental.pallas import tpu as pltpu
from jax.experimental.pallas import tpu_sc as plsc

DIM = 64
LN = 32
LP = 32
LE = 16
NC = 2
NS = 16
NW = NC * NS
QUAD = 4
NBUF = 2

_LN2 = math.log(2.0)

_GATHER_DNUMS = lax.GatherDimensionNumbers(
    offset_dims=(), collapsed_slice_dims=(0,), start_index_map=(0,))


def _shuffle(v, idx):
    return lax.gather(v, idx[:, None], _GATHER_DNUMS, slice_sizes=(1,),
                      mode=lax.GatherScatterMode.PROMISE_IN_BOUNDS)


def _lane_sum(v):
    for sh in (8, 4, 2, 1):
        idx = lax.iota(jnp.int32, 16) ^ sh
        v = v + _shuffle(v, idx)
    return v


def _rsqrt(x):
    i = plsc.bitcast(x, jnp.int32)
    y = plsc.bitcast(jnp.int32(0x5F3759DF) - lax.shift_right_arithmetic(i, 1),
                     jnp.float32)
    for _ in range(3):
        y = y * (jnp.float32(1.5) - jnp.float32(0.5) * x * y * y)
    return y


def _log(x):
    i = plsc.bitcast(x, jnp.int32)
    f = i.astype(jnp.float32)
    y = f * jnp.float32(_LN2 / (1 << 23)) - jnp.float32(126.94269504 * _LN2)
    for _ in range(3):
        y = y + x * jnp.exp(-y) - jnp.float32(1.0)
    return y


_LANE = None


def _ctx(row_load, L, si, unroll):
    assert L % 4 == 0
    s0, s1, s2, s3 = si
    q0 = s0 * s0 + s1 * s1 + s2 * s2 + s3 * s3

    lane = lax.iota(jnp.int32, 16)
    mq = (lane & 4) == 0
    mh = lane < 8
    ix8 = lane ^ 8
    ix4 = lane ^ 4
    ix2 = lane ^ 2
    ix1 = lane ^ 1

    def p1(i, carry):
        a0, a1, a2, a3, qa = carry
        rows = [row_load(i * 4 + u) for u in range(4)]
        t = [(rows[0][k] + rows[1][k]) + (rows[2][k] + rows[3][k])
             for k in range(4)]
        sq = [((r[0] * r[0] + r[1] * r[1]) + (r[2] * r[2] + r[3] * r[3]))
              for r in rows]
        return (a0 + t[0], a1 + t[1], a2 + t[2], a3 + t[3],
                qa + ((sq[0] + sq[1]) + (sq[2] + sq[3])))

    a0, a1, a2, a3, qa = lax.fori_loop(0, L // 4, p1, (s0, s1, s2, s3, q0),
                                       unroll=unroll)
    q = _lane_sum(qa)
    rinv = _rsqrt(q)
    g0, g1, g2, g3 = a0 * rinv, a1 * rinv, a2 * rinv, a3 * rinv
    sc0 = _lane_sum(s0 * g0 + s1 * g1 + s2 * g2 + s3 * g3)
    sum_sc = _lane_sum(a0 * g0 + a1 * g1 + a2 * g2 + a3 * g3)

    def p2(i, se):
        hs = []
        for u in range(4):
            r0, r1, r2, r3 = row_load(i * 4 + u)
            p = (r0 * g0 + r1 * g1) + (r2 * g2 + r3 * g3)
            h = p + _shuffle(p, ix8)
            h = h + _shuffle(h, ix4)
            hs.append(h)
        ab = jnp.where(mq, hs[0], hs[1])
        cd = jnp.where(mq, hs[2], hs[3])
        qv = jnp.where(mh, ab, cd)
        qv = qv + _shuffle(qv, ix2)
        qv = qv + _shuffle(qv, ix1)
        return se + jnp.exp(qv)

    se4 = lax.fori_loop(0, L // 4, p2, jnp.zeros((16,), jnp.float32),
                        unroll=unroll)
    se = jnp.float32(0.25) * _lane_sum(se4) + jnp.exp(sc0)
    lse = _log(se)
    return sum_sc - jnp.float32(L + 1) * lse, sc0 - lse


@functools.lru_cache(maxsize=None)
def _build(B):
    assert B % NW == 0
    SPW = B // NW
    NQ = SPW // QUAD

    mesh = plsc.VectorSubcoreMesh(core_axis_name="c", subcore_axis_name="s",
                                  num_cores=NC, num_subcores=NS)

    @functools.partial(
        pl.kernel,
        out_type=jax.ShapeDtypeStruct((B,), jnp.float32),
        mesh=mesh,
        compiler_params=pltpu.CompilerParams(needs_layout_passes=False,
                                             use_tc_tiling_on_sc=False),
        scratch_types=[
            pltpu.VMEM((SPW,), jnp.int32),
            pltpu.VMEM((SPW * LN,), jnp.int32),
            pltpu.VMEM((SPW * LP,), jnp.int32),
            pltpu.VMEM((SPW * LE,), jnp.int32),
            pltpu.VMEM((SPW, DIM), jnp.float32),
            pltpu.VMEM((NBUF * QUAD * LN, DIM), jnp.float32),
            pltpu.VMEM((NBUF * QUAD * LP, DIM), jnp.float32),
            pltpu.VMEM((NBUF * QUAD * LE, DIM), jnp.float32),
            pltpu.VMEM((SPW,), jnp.float32),
            pltpu.SemaphoreType.DMA,
            pltpu.SemaphoreType.DMA,
        ],
    )
    def sc_kernel(nid_h, nbr_h, pth_h, edg_h, ent_h, rel_h, out_h,
                  nid_v, nbr_v, pth_v, edg_v, si_all, n_r, p_r, e_r,
                  out_v, sem0, sem1):
        c = lax.axis_index("c")
        s = lax.axis_index("s")
        wid = s * NC + c
        base = wid * SPW

        pltpu.sync_copy(nid_h.at[pl.ds(base, SPW)], nid_v)
        pltpu.sync_copy(nbr_h.at[pl.ds(base * LN, SPW * LN)], nbr_v)
        pltpu.sync_copy(pth_h.at[pl.ds(base * LP, SPW * LP)], pth_v)
        pltpu.sync_copy(edg_h.at[pl.ds(base * LE, SPW * LE)], edg_v)

        pltpu.async_copy(ent_h.at[nid_v], si_all, sem0).wait()

        def dmas(q, slot, sem):
            return (
                pltpu.make_async_copy(
                    ent_h.at[nbr_v.at[pl.ds(q * (QUAD * LN), QUAD * LN)]],
                    n_r.at[pl.ds(slot * (QUAD * LN), QUAD * LN)], sem),
                pltpu.make_async_copy(
                    ent_h.at[pth_v.at[pl.ds(q * (QUAD * LP), QUAD * LP)]],
                    p_r.at[pl.ds(slot * (QUAD * LP), QUAD * LP)], sem),
                pltpu.make_async_copy(
                    rel_h.at[edg_v.at[pl.ds(q * (QUAD * LE), QUAD * LE)]],
                    e_r.at[pl.ds(slot * (QUAD * LE), QUAD * LE)], sem),
            )

        def issue(q, slot, sem):
            for d in dmas(q, slot, sem):
                d.start()

        def drain(q, slot, sem):
            for d in dmas(q, slot, sem):
                d.wait()

        issue(0, 0, sem0)

        lane = lax.iota(jnp.int32, 16)
        lane0 = lane == 0

        def quad(q, _):
            par = jnp.bitwise_and(q, 1)
            cur = par
            nxt = 1 - par

            @pl.when(jnp.logical_and(q + 1 < NQ, par == 0))
            def _():
                issue(q + 1, 1, sem1)

            @pl.when(jnp.logical_and(q + 1 < NQ, par == 1))
            def _():
                issue(q + 1, 0, sem0)

            @pl.when(par == 0)
            def _():
                drain(q, 0, sem0)

            @pl.when(par == 1)
            def _():
                drain(q, 1, sem1)

            slot0 = cur * QUAD
            for j in range(QUAD):
                t = q * QUAD + j
                nb = (slot0 + j) * LN
                pb = (slot0 + j) * LP
                eb = (slot0 + j) * LE
                si = tuple(si_all[t, pl.ds(16 * k, 16)] for k in range(4))
                n_acc, _ = _ctx(
                    lambda i: tuple(n_r[nb + i, pl.ds(16 * k, 16)]
                                    for k in range(4)), LN, si, 2)
                p_acc, _ = _ctx(
                    lambda i: tuple(p_r[pb + i, pl.ds(16 * k, 16)]
                                    for k in range(4)), LP, si, 2)
                _, e_first = _ctx(
                    lambda i: tuple(e_r[eb + i, pl.ds(16 * k, 16)]
                                    for k in range(4)), LE, si, 2)
                loss = -(n_acc + jnp.float32(0.1) * p_acc
                         + jnp.float32(0.1) * e_first)
                idx = lax.broadcast(t, (16,)).astype(jnp.int32)
                plsc.store_scatter(out_v, [idx], loss, mask=lane0)
            return 0

        lax.fori_loop(0, NQ, quad, 0)
        pltpu.sync_copy(out_v, out_h.at[pl.ds(base, SPW)])

    return sc_kernel


def kernel(node_ids, neighbor_ids, path_ids, edge_ids, ent_table, rel_table):
    B = node_ids.shape[0]
    f = _build(B)
    return f(node_ids.astype(jnp.int32),
             neighbor_ids.astype(jnp.int32).reshape(-1),
             path_ids.astype(jnp.int32).reshape(-1),
             edge_ids.astype(jnp.int32).reshape(-1),
             ent_table, rel_table)
